# Initial kernel scaffold; baseline (speedup 1.0000x reference)
import jax
import jax.numpy as jnp
from jax.experimental import pallas as pl


def kernel(x, edge_index, batch, W1, b1, W2, b2, W3, b3):
    raise NotImplementedError("write your pallas kernel here")


# speedup vs baseline: 20.9504x; 20.9504x over previous
import jax
import jax.numpy as jnp
from jax import lax
from jax.experimental import pallas as pl
from jax.experimental.pallas import tpu as pltpu
from jax.experimental.pallas import tpu_sc as plsc

N = 10000
NP = 10240
E = 320000
D = 128
G = 64
NSC = 2
NTILE = 16
NW = NSC * NTILE
EPW = E // NW
CHUNK = 128
NCH = 80
ROWS_PT = NP // NTILE
BLK = 1024
NBLK = NP // BLK

_mesh = plsc.VectorSubcoreMesh(core_axis_name="c", subcore_axis_name="s")


def _deg_body(dstp, zdeg, onesb, deg_out, deg_sh, dst_v, ones_v):
    c = lax.axis_index("c")
    s = lax.axis_index("s")
    w = c * NTILE + s
    pltpu.sync_copy(dstp.at[w], dst_v)
    pltpu.sync_copy(onesb, ones_v)
    r0 = s * ROWS_PT
    pltpu.sync_copy(zdeg.at[pl.ds(r0, ROWS_PT)], deg_sh.at[pl.ds(r0, ROWS_PT)])
    plsc.subcore_barrier()

    def body(j, carry):
        pltpu.sync_copy(ones_v, deg_sh.at[dst_v.at[j]], add=True)
        return carry

    lax.fori_loop(0, NCH, body, 0, unroll=4)
    plsc.subcore_barrier()
    pltpu.sync_copy(deg_sh.at[pl.ds(r0, ROWS_PT)], deg_out.at[c, pl.ds(r0, ROWS_PT)])


_deg_call = pl.kernel(
    _deg_body,
    out_type=jax.ShapeDtypeStruct((NSC, NP), jnp.float32),
    mesh=_mesh,
    scratch_types=[
        pltpu.VMEM_SHARED((NP,), jnp.float32),
        pltpu.VMEM((NCH, CHUNK), jnp.int32),
        pltpu.VMEM((CHUNK,), jnp.float32),
    ],
)


WCH = 8
NWIN = NCH // WCH


def _edge_body(g_hbm, srcp, dstp, zinit, acc_out,
               acc_sh, src_w, dst_w, buf0, buf1, sem0, sem1):
    c = lax.axis_index("c")
    s = lax.axis_index("s")
    w = c * NTILE + s
    r0 = s * ROWS_PT
    pltpu.sync_copy(zinit.at[pl.ds(r0, ROWS_PT)], acc_sh.at[pl.ds(r0, ROWS_PT)])
    plsc.subcore_barrier()

    def window(wi, carry):
        pltpu.sync_copy(srcp.at[w, pl.ds(wi * WCH, WCH)], src_w)
        pltpu.sync_copy(dstp.at[w, pl.ds(wi * WCH, WCH)], dst_w)
        for p in range(WCH // 2):
            a = 2 * p
            cp0 = pltpu.async_copy(g_hbm.at[src_w.at[a]], buf0, sem0)
            cp1 = pltpu.async_copy(g_hbm.at[src_w.at[a + 1]], buf1, sem1)
            cp0.wait()
            pltpu.sync_copy(buf0, acc_sh.at[dst_w.at[a]], add=True)
            cp1.wait()
            pltpu.sync_copy(buf1, acc_sh.at[dst_w.at[a + 1]], add=True)
        return carry

    lax.fori_loop(0, NWIN, window, 0)
    plsc.subcore_barrier()
    pltpu.sync_copy(acc_sh.at[pl.ds(r0, ROWS_PT)],
                    acc_out.at[c, pl.ds(r0, ROWS_PT)])


_edge_call = pl.kernel(
    _edge_body,
    out_type=jax.ShapeDtypeStruct((NSC, NP, D), jnp.float32),
    mesh=_mesh,
    scratch_types=[
        pltpu.VMEM_SHARED((NP, D), jnp.float32),
        pltpu.VMEM((WCH, CHUNK), jnp.int32),
        pltpu.VMEM((WCH, CHUNK), jnp.int32),
        pltpu.VMEM((CHUNK, D), jnp.float32),
        pltpu.VMEM((CHUNK, D), jnp.float32),
        pltpu.SemaphoreType.DMA,
        pltpu.SemaphoreType.DMA,
    ],
)


def _tc1_body(deg_ref, x_ref, w_ref, g_ref, dinv_ref):
    deg = deg_ref[...]
    dinv = lax.rsqrt(deg[0:1] + deg[1:2] + 1.0)
    dinvc = dinv.reshape(BLK, 1)
    g_ref[...] = dinvc * jnp.dot(x_ref[...], w_ref[...],
                                 preferred_element_type=jnp.float32)
    dinv_ref[...] = dinv.reshape(1, 1, BLK)


def _tc1(deg, xp, W1):
    return pl.pallas_call(
        _tc1_body,
        grid=(NBLK,),
        in_specs=[
            pl.BlockSpec((NSC, BLK), lambda i: (0, i)),
            pl.BlockSpec((BLK, D), lambda i: (i, 0)),
            pl.BlockSpec((D, D), lambda i: (0, 0)),
        ],
        out_specs=[
            pl.BlockSpec((BLK, D), lambda i: (i, 0)),
            pl.BlockSpec((1, 1, BLK), lambda i: (i, 0, 0)),
        ],
        out_shape=[
            jax.ShapeDtypeStruct((NP, D), jnp.float32),
            jax.ShapeDtypeStruct((NBLK, 1, BLK), jnp.float32),
        ],
    )(deg, xp, W1)


def _mask_from_batch(batch_row):
    gid = lax.broadcasted_iota(jnp.int32, (G, BLK), 0)
    return (gid == batch_row).astype(jnp.float32)


def _tc_mid_body(g_ref, acc_ref, dinv_ref, b_ref, w_ref, batch_ref,
                 gn_ref, pool_ref):
    i = pl.program_id(0)
    acc = acc_ref[...]
    dinvc = dinv_ref[...].reshape(BLK, 1)
    h = dinvc * (g_ref[...] + acc[0] + acc[1]) + b_ref[...]
    h = jnp.where(h > 0, h, 0.01 * h)
    gn_ref[...] = dinvc * jnp.dot(h, w_ref[...],
                                  preferred_element_type=jnp.float32)
    mf = _mask_from_batch(batch_ref[...].reshape(1, BLK))

    @pl.when(i == 0)
    def _():
        pool_ref[...] = jnp.zeros_like(pool_ref)

    pool_ref[...] += jnp.dot(mf, h, preferred_element_type=jnp.float32)


def _tc_mid(g, acc, dinv, b, Wn, batchp):
    return pl.pallas_call(
        _tc_mid_body,
        grid=(NBLK,),
        in_specs=[
            pl.BlockSpec((BLK, D), lambda i: (i, 0)),
            pl.BlockSpec((NSC, BLK, D), lambda i: (0, i, 0)),
            pl.BlockSpec((1, 1, BLK), lambda i: (i, 0, 0)),
            pl.BlockSpec((1, D), lambda i: (0, 0)),
            pl.BlockSpec((D, D), lambda i: (0, 0)),
            pl.BlockSpec((1, 1, BLK), lambda i: (i, 0, 0)),
        ],
        out_specs=[
            pl.BlockSpec((BLK, D), lambda i: (i, 0)),
            pl.BlockSpec((G, D), lambda i: (0, 0)),
        ],
        out_shape=[
            jax.ShapeDtypeStruct((NP, D), jnp.float32),
            jax.ShapeDtypeStruct((G, D), jnp.float32),
        ],
    )(g, acc, dinv, b, Wn, batchp)


def _tc_final_body(g_ref, acc_ref, dinv_ref, b_ref, batch_ref, p1_ref, p2_ref,
                   merge_ref, pool_s, cnt_s):
    i = pl.program_id(0)
    acc = acc_ref[...]
    dinvc = dinv_ref[...].reshape(BLK, 1)
    h = dinvc * (g_ref[...] + acc[0] + acc[1]) + b_ref[...]
    h = jnp.where(h > 0, h, 0.01 * h)
    mf = _mask_from_batch(batch_ref[...].reshape(1, BLK))

    @pl.when(i == 0)
    def _():
        pool_s[...] = jnp.zeros_like(pool_s)
        cnt_s[...] = jnp.zeros_like(cnt_s)

    pool_s[...] += jnp.dot(mf, h, preferred_element_type=jnp.float32)
    cnt_s[...] += jnp.broadcast_to(
        jnp.sum(mf, axis=1, keepdims=True), (G, D))

    @pl.when(i == NBLK - 1)
    def _():
        tot = p1_ref[...] + p2_ref[...] + pool_s[...]
        merge_ref[...] = tot / (3.0 * jnp.maximum(cnt_s[...], 1.0))


def _tc_final(g, acc, dinv, b, batchp, p1, p2):
    return pl.pallas_call(
        _tc_final_body,
        grid=(NBLK,),
        in_specs=[
            pl.BlockSpec((BLK, D), lambda i: (i, 0)),
            pl.BlockSpec((NSC, BLK, D), lambda i: (0, i, 0)),
            pl.BlockSpec((1, 1, BLK), lambda i: (i, 0, 0)),
            pl.BlockSpec((1, D), lambda i: (0, 0)),
            pl.BlockSpec((1, 1, BLK), lambda i: (i, 0, 0)),
            pl.BlockSpec((G, D), lambda i: (0, 0)),
            pl.BlockSpec((G, D), lambda i: (0, 0)),
        ],
        out_specs=pl.BlockSpec((G, D), lambda i: (0, 0)),
        out_shape=jax.ShapeDtypeStruct((G, D), jnp.float32),
        scratch_shapes=[
            pltpu.VMEM((G, D), jnp.float32),
            pltpu.VMEM((G, D), jnp.float32),
        ],
    )(g, acc, dinv, b, batchp, p1, p2)


def kernel(x, edge_index, batch, W1, b1, W2, b2, W3, b3):
    src = edge_index[0].reshape(NW, EPW)
    dst = edge_index[1].reshape(NW, EPW)
    padn = NCH * CHUNK - EPW
    trash = N + (jnp.arange(padn, dtype=jnp.int32) % (NP - N))
    srcp = jnp.concatenate(
        [src, jnp.broadcast_to(trash, (NW, padn))], axis=1).reshape(NW, NCH, CHUNK)
    dstp = jnp.concatenate(
        [dst, jnp.broadcast_to(trash, (NW, padn))], axis=1).reshape(NW, NCH, CHUNK)
    xp = jnp.concatenate([x, jnp.zeros((NP - N, D), x.dtype)])
    batchp = jnp.concatenate(
        [batch, jnp.full((NP - N,), G, jnp.int32)]).reshape(NBLK, 1, BLK)
    zinit = jnp.zeros((NP, D), jnp.float32)
    zdeg = jnp.zeros((NP,), jnp.float32)
    onesb = jnp.ones((CHUNK,), jnp.float32)
    b1r, b2r, b3r = (b.reshape(1, D) for b in (b1, b2, b3))

    deg = _deg_call(dstp, zdeg, onesb)
    g1, dinv = _tc1(deg, xp, W1)
    acc1 = _edge_call(g1, srcp, dstp, zinit)
    g2, p1 = _tc_mid(g1, acc1, dinv, b1r, W2, batchp)
    acc2 = _edge_call(g2, srcp, dstp, zinit)
    g3, p2 = _tc_mid(g2, acc2, dinv, b2r, W3, batchp)
    acc3 = _edge_call(g3, srcp, dstp, zinit)
    merge = _tc_final(g3, acc3, dinv, b3r, batchp, p1, p2)
    return (merge, 0)


# speedup vs baseline: 27.7659x; 1.3253x over previous
import jax
import jax.numpy as jnp
from jax import lax
from jax.experimental import pallas as pl
from jax.experimental.pallas import tpu as pltpu
from jax.experimental.pallas import tpu_sc as plsc

N = 10000
NP = 10240
E = 320000
D = 128
G = 64
NSC = 2
NTILE = 16
NW = NSC * NTILE
EPW = E // NW
CHUNK = 128
WCH = 8
NWIN = 10
NCH = WCH * NWIN
RPT = NP // NTILE
BLK = 1024
NBLK = NP // BLK

_mesh = plsc.VectorSubcoreMesh(core_axis_name="c", subcore_axis_name="s")


def _deg_body(dstp, zdeg, onesb, deg_out, deg_sh, dst_v, ones_v):
    c = lax.axis_index("c")
    s = lax.axis_index("s")
    w = c * NTILE + s
    pltpu.sync_copy(dstp.at[w], dst_v)
    pltpu.sync_copy(onesb, ones_v)
    r0 = s * RPT
    pltpu.sync_copy(zdeg.at[pl.ds(r0, RPT)], deg_sh.at[pl.ds(r0, RPT)])
    plsc.subcore_barrier()

    def body(j, carry):
        pltpu.sync_copy(ones_v, deg_sh.at[dst_v.at[j]], add=True)
        return carry

    lax.fori_loop(0, NCH, body, 0, unroll=4)
    plsc.subcore_barrier()
    pltpu.sync_copy(deg_sh.at[pl.ds(r0, RPT)], deg_out.at[c, pl.ds(r0, RPT)])


_deg_call = pl.kernel(
    _deg_body,
    out_type=jax.ShapeDtypeStruct((NSC, NP), jnp.float32),
    mesh=_mesh,
    scratch_types=[
        pltpu.VMEM_SHARED((NP,), jnp.float32),
        pltpu.VMEM((NCH, CHUNK), jnp.int32),
        pltpu.VMEM((CHUNK,), jnp.float32),
    ],
)


def _edge_body(g_hbm, srcp, dstp, zinit, acc_out,
               acc_sh, src_w, dst_w, bufs, gsems, ssems):
    c = lax.axis_index("c")
    s = lax.axis_index("s")
    w = c * NTILE + s
    r0 = s * RPT
    pltpu.sync_copy(zinit.at[pl.ds(r0, RPT)], acc_sh.at[pl.ds(r0, RPT)])
    plsc.subcore_barrier()

    pltpu.sync_copy(srcp.at[w, pl.ds(0, WCH)], src_w.at[0])
    pltpu.sync_copy(dstp.at[w, pl.ds(0, WCH)], dst_w.at[0])
    pltpu.async_copy(g_hbm.at[src_w.at[0, 0]], bufs.at[0], gsems.at[0])
    pltpu.async_copy(g_hbm.at[src_w.at[0, 1]], bufs.at[1], gsems.at[1])

    def window(wi, carry):
        p = wi % 2
        pn = (wi + 1) % 2
        for k in range(WCH):
            b = k % 2
            pltpu.make_async_copy(
                g_hbm.at[src_w.at[p, k]], bufs.at[b], gsems.at[b]).wait()
            pltpu.async_copy(
                bufs.at[b], acc_sh.at[dst_w.at[p, k]], ssems.at[b], add=True)
            if k == WCH - 3:
                @pl.when(wi < NWIN - 1)
                def _():
                    pltpu.sync_copy(
                        srcp.at[w, pl.ds((wi + 1) * WCH, WCH)], src_w.at[pn])
                    pltpu.sync_copy(
                        dstp.at[w, pl.ds((wi + 1) * WCH, WCH)], dst_w.at[pn])
            pltpu.make_async_copy(
                bufs.at[b], acc_sh.at[dst_w.at[p, k]], ssems.at[b]).wait()
            if k < WCH - 2:
                pltpu.async_copy(
                    g_hbm.at[src_w.at[p, k + 2]], bufs.at[b], gsems.at[b])
            else:
                @pl.when(wi < NWIN - 1)
                def _():
                    pltpu.async_copy(
                        g_hbm.at[src_w.at[pn, k - (WCH - 2)]],
                        bufs.at[b], gsems.at[b])
        return carry

    lax.fori_loop(0, NWIN, window, 0)
    plsc.subcore_barrier()
    pltpu.sync_copy(acc_sh.at[pl.ds(r0, RPT)], acc_out.at[c, pl.ds(r0, RPT)])


_edge_call = pl.kernel(
    _edge_body,
    out_type=jax.ShapeDtypeStruct((NSC, NP, D), jnp.float32),
    mesh=_mesh,
    scratch_types=[
        pltpu.VMEM_SHARED((NP, D), jnp.float32),
        pltpu.VMEM((2, WCH, CHUNK), jnp.int32),
        pltpu.VMEM((2, WCH, CHUNK), jnp.int32),
        pltpu.VMEM((2, CHUNK, D), jnp.float32),
        pltpu.SemaphoreType.DMA((2,)),
        pltpu.SemaphoreType.DMA((2,)),
    ],
)


def _tc1_body(deg_ref, x_ref, w_ref, g_ref, dinv_ref):
    deg = deg_ref[...]
    dinv = lax.rsqrt(deg[0, 0] + deg[1, 0] + 1.0)
    dinvc = dinv.reshape(BLK, 1)
    g_ref[...] = dinvc * jnp.dot(x_ref[...], w_ref[...],
                                 preferred_element_type=jnp.float32)
    dinv_ref[...] = dinv.reshape(1, 1, BLK)


def _tc1(deg4, xp, W1):
    return pl.pallas_call(
        _tc1_body,
        grid=(NBLK,),
        in_specs=[
            pl.BlockSpec((NSC, 1, 1, BLK), lambda i: (0, i, 0, 0)),
            pl.BlockSpec((BLK, D), lambda i: (i, 0)),
            pl.BlockSpec((D, D), lambda i: (0, 0)),
        ],
        out_specs=[
            pl.BlockSpec((BLK, D), lambda i: (i, 0)),
            pl.BlockSpec((1, 1, BLK), lambda i: (i, 0, 0)),
        ],
        out_shape=[
            jax.ShapeDtypeStruct((NP, D), jnp.float32),
            jax.ShapeDtypeStruct((NBLK, 1, BLK), jnp.float32),
        ],
    )(deg4, xp, W1)


def _mask_from_batch(batch_row):
    gid = lax.broadcasted_iota(jnp.int32, (G, BLK), 0)
    return (gid == batch_row).astype(jnp.float32)


def _tc_mid_body(g_ref, acc_ref, dinv_ref, b_ref, w_ref, batch_ref,
                 gn_ref, pool_ref):
    i = pl.program_id(0)
    acc = acc_ref[...]
    dinvc = dinv_ref[...].reshape(BLK, 1)
    h = dinvc * (g_ref[...] + acc[0] + acc[1]) + b_ref[...]
    h = jnp.where(h > 0, h, 0.01 * h)
    gn_ref[...] = dinvc * jnp.dot(h, w_ref[...],
                                  preferred_element_type=jnp.float32)
    mf = _mask_from_batch(batch_ref[...].reshape(1, BLK))

    @pl.when(i == 0)
    def _():
        pool_ref[...] = jnp.zeros_like(pool_ref)

    pool_ref[...] += jnp.dot(mf, h, preferred_element_type=jnp.float32)


def _tc_mid(g, acc, dinv, b, Wn, batchp):
    return pl.pallas_call(
        _tc_mid_body,
        grid=(NBLK,),
        in_specs=[
            pl.BlockSpec((BLK, D), lambda i: (i, 0)),
            pl.BlockSpec((NSC, BLK, D), lambda i: (0, i, 0)),
            pl.BlockSpec((1, 1, BLK), lambda i: (i, 0, 0)),
            pl.BlockSpec((1, D), lambda i: (0, 0)),
            pl.BlockSpec((D, D), lambda i: (0, 0)),
            pl.BlockSpec((1, 1, BLK), lambda i: (i, 0, 0)),
        ],
        out_specs=[
            pl.BlockSpec((BLK, D), lambda i: (i, 0)),
            pl.BlockSpec((G, D), lambda i: (0, 0)),
        ],
        out_shape=[
            jax.ShapeDtypeStruct((NP, D), jnp.float32),
            jax.ShapeDtypeStruct((G, D), jnp.float32),
        ],
    )(g, acc, dinv, b, Wn, batchp)


def _tc_final_body(g_ref, acc_ref, dinv_ref, b_ref, batch_ref, p1_ref, p2_ref,
                   merge_ref, pool_s, cnt_s):
    i = pl.program_id(0)
    acc = acc_ref[...]
    dinvc = dinv_ref[...].reshape(BLK, 1)
    h = dinvc * (g_ref[...] + acc[0] + acc[1]) + b_ref[...]
    h = jnp.where(h > 0, h, 0.01 * h)
    mf = _mask_from_batch(batch_ref[...].reshape(1, BLK))

    @pl.when(i == 0)
    def _():
        pool_s[...] = jnp.zeros_like(pool_s)
        cnt_s[...] = jnp.zeros_like(cnt_s)

    pool_s[...] += jnp.dot(mf, h, preferred_element_type=jnp.float32)
    cnt_s[...] += jnp.broadcast_to(
        jnp.sum(mf, axis=1, keepdims=True), (G, D))

    @pl.when(i == NBLK - 1)
    def _():
        tot = p1_ref[...] + p2_ref[...] + pool_s[...]
        merge_ref[...] = tot / (3.0 * jnp.maximum(cnt_s[...], 1.0))


def _tc_final(g, acc, dinv, b, batchp, p1, p2):
    return pl.pallas_call(
        _tc_final_body,
        grid=(NBLK,),
        in_specs=[
            pl.BlockSpec((BLK, D), lambda i: (i, 0)),
            pl.BlockSpec((NSC, BLK, D), lambda i: (0, i, 0)),
            pl.BlockSpec((1, 1, BLK), lambda i: (i, 0, 0)),
            pl.BlockSpec((1, D), lambda i: (0, 0)),
            pl.BlockSpec((1, 1, BLK), lambda i: (i, 0, 0)),
            pl.BlockSpec((G, D), lambda i: (0, 0)),
            pl.BlockSpec((G, D), lambda i: (0, 0)),
        ],
        out_specs=pl.BlockSpec((G, D), lambda i: (0, 0)),
        out_shape=jax.ShapeDtypeStruct((G, D), jnp.float32),
        scratch_shapes=[
            pltpu.VMEM((G, D), jnp.float32),
            pltpu.VMEM((G, D), jnp.float32),
        ],
    )(g, acc, dinv, b, batchp, p1, p2)


def kernel(x, edge_index, batch, W1, b1, W2, b2, W3, b3):
    src = edge_index[0].reshape(NW, EPW)
    dst = edge_index[1].reshape(NW, EPW)
    padn = NCH * CHUNK - EPW
    trash = N + (jnp.arange(padn, dtype=jnp.int32) % (NP - N))
    srcp = jnp.concatenate(
        [src, jnp.broadcast_to(trash, (NW, padn))], axis=1).reshape(NW, NCH, CHUNK)
    dstp = jnp.concatenate(
        [dst, jnp.broadcast_to(trash, (NW, padn))], axis=1).reshape(NW, NCH, CHUNK)
    xp = jnp.concatenate([x, jnp.zeros((NP - N, D), x.dtype)])
    batchp = jnp.concatenate(
        [batch, jnp.full((NP - N,), G, jnp.int32)]).reshape(NBLK, 1, BLK)
    zinit = jnp.zeros((NP, D), jnp.float32)
    zdeg = jnp.zeros((NP,), jnp.float32)
    onesb = jnp.ones((CHUNK,), jnp.float32)
    b1r, b2r, b3r = (b.reshape(1, D) for b in (b1, b2, b3))

    deg = _deg_call(dstp, zdeg, onesb)
    deg4 = deg.reshape(NSC, NBLK, 1, BLK)
    g1, dinv = _tc1(deg4, xp, W1)
    acc1 = _edge_call(g1, srcp, dstp, zinit)
    g2, p1 = _tc_mid(g1, acc1, dinv, b1r, W2, batchp)
    acc2 = _edge_call(g2, srcp, dstp, zinit)
    g3, p2 = _tc_mid(g2, acc2, dinv, b2r, W3, batchp)
    acc3 = _edge_call(g3, srcp, dstp, zinit)
    merge = _tc_final(g3, acc3, dinv, b3r, batchp, p1, p2)
    return (merge, 0)
